# baseline (speedup 1.0000x reference)
import jax
import jax.numpy as jnp
from jax import lax
from jax.experimental import pallas as pl
from jax.experimental.pallas import tpu as pltpu
from jax.experimental.pallas import tpu_sc as plsc

_N = 10000
_E = 320000
_D = 128
_NPAD = 10240
_NC = 2
_NS = 16
_NW = _NC * _NS
_K = 125
_CH = _E // (_NW * _K)
_RPT = _NPAD // _NS

_BLK = 1000
_GRID = _N // _BLK


def _sc_mesh():
    return plsc.VectorSubcoreMesh(core_axis_name="c", subcore_axis_name="s")


def _deg_body(dst_hbm, ones_hbm, zeros_hbm, out_hbm, dst_v, ones_v, deg_sh):
    cid = lax.axis_index("c")
    sid = lax.axis_index("s")
    wid = sid * _NC + cid
    pltpu.sync_copy(dst_hbm.at[wid], dst_v)
    pltpu.sync_copy(ones_hbm, ones_v)
    rows = pl.ds(sid * _RPT, _RPT)
    pltpu.sync_copy(zeros_hbm.at[rows], deg_sh.at[rows])
    plsc.subcore_barrier()

    @pl.loop(0, _CH)
    def _chunk(j):
        pltpu.sync_copy(ones_v, deg_sh.at[dst_v.at[j]], add=True)

    plsc.subcore_barrier()
    pltpu.sync_copy(deg_sh.at[rows], out_hbm.at[cid, rows])


def _make_deg():
    return pl.kernel(
        _deg_body,
        out_type=jax.ShapeDtypeStruct((_NC, _NPAD), jnp.float32),
        mesh=_sc_mesh(),
        scratch_types=[
            pltpu.VMEM((_CH, _K), jnp.int32),
            pltpu.VMEM((_K,), jnp.float32),
            pltpu.VMEM_SHARED((_NPAD,), jnp.float32),
        ],
    )


def _agg_body(idx_hbm, hs_hbm, out_hbm,
              ibuf_a, ibuf_b, rows_a, rows_b,
              agg_sh, sem_ia, sem_ib, sem_ga, sem_gb):
    cid = lax.axis_index("c")
    sid = lax.axis_index("s")
    wid = sid * _NC + cid
    rows = pl.ds(sid * _RPT, _RPT)

    zv = jnp.zeros((16,), jnp.float32)

    @pl.loop(0, _K)
    def _z(i):
        for c in range(_D // 16):
            rows_a[i, pl.ds(c * 16, 16)] = zv
            rows_b[i, pl.ds(c * 16, 16)] = zv

    base = sid * _RPT

    @pl.when(cid == 0)
    def _():
        @pl.when(sid < _NS - 1)
        def _():
            pltpu.sync_copy(hs_hbm.at[pl.ds(base, _RPT)], agg_sh.at[rows])

        @pl.when(sid == _NS - 1)
        def _():
            pltpu.sync_copy(hs_hbm.at[pl.ds(9600, 400)],
                            agg_sh.at[pl.ds(9600, 400)])
            pltpu.sync_copy(rows_a, agg_sh.at[pl.ds(10000, _K)])
            pltpu.sync_copy(rows_b.at[pl.ds(0, 115)],
                            agg_sh.at[pl.ds(10125, 115)])

    @pl.when(cid != 0)
    def _():
        @pl.loop(0, 5)
        def _f(q):
            pltpu.sync_copy(rows_a, agg_sh.at[pl.ds(base + q * _K, _K)])

        pltpu.sync_copy(rows_b.at[pl.ds(0, 15)],
                        agg_sh.at[pl.ds(base + 5 * _K, 15)])

    plsc.subcore_barrier()

    def start_idx(pair, ibuf, sem):
        pltpu.async_copy(idx_hbm.at[wid, pair], ibuf, sem)

    def wait_idx(pair, ibuf, sem):
        pltpu.make_async_copy(idx_hbm.at[wid, pair], ibuf, sem).wait()

    def start_gather(ibuf, half, buf, sem):
        pltpu.async_copy(hs_hbm.at[ibuf.at[half, 0]], buf, sem)

    def wait_gather(ibuf, half, buf, sem):
        pltpu.make_async_copy(hs_hbm.at[ibuf.at[half, 0]], buf, sem).wait()

    def scatter(buf, ibuf, half):
        pltpu.sync_copy(buf, agg_sh.at[ibuf.at[half, 1]], add=True)

    pltpu.sync_copy(idx_hbm.at[wid, 0], ibuf_a)
    start_gather(ibuf_a, 0, rows_a, sem_ga)
    start_idx(1, ibuf_b, sem_ib)

    @pl.loop(0, _CH, step=4)
    def _quad(j):
        p = j // 4
        wait_gather(ibuf_a, 0, rows_a, sem_ga)
        start_gather(ibuf_a, 1, rows_b, sem_gb)
        scatter(rows_a, ibuf_a, 0)
        wait_idx(2 * p + 1, ibuf_b, sem_ib)
        start_gather(ibuf_b, 0, rows_a, sem_ga)
        wait_gather(ibuf_a, 1, rows_b, sem_gb)
        scatter(rows_b, ibuf_a, 1)

        @pl.when(j + 4 < _CH)
        def _():
            start_idx(2 * p + 2, ibuf_a, sem_ia)

        wait_gather(ibuf_b, 0, rows_a, sem_ga)
        start_gather(ibuf_b, 1, rows_b, sem_gb)
        scatter(rows_a, ibuf_b, 0)

        @pl.when(j + 4 < _CH)
        def _():
            wait_idx(2 * p + 2, ibuf_a, sem_ia)
            start_gather(ibuf_a, 0, rows_a, sem_ga)

        wait_gather(ibuf_b, 1, rows_b, sem_gb)
        scatter(rows_b, ibuf_b, 1)

        @pl.when(j + 6 < _CH)
        def _():
            start_idx(2 * p + 3, ibuf_b, sem_ib)

    plsc.subcore_barrier()
    pltpu.sync_copy(agg_sh.at[rows], out_hbm.at[cid, rows])


def _make_agg():
    return pl.kernel(
        _agg_body,
        out_type=jax.ShapeDtypeStruct((_NC, _NPAD, _D), jnp.float32),
        mesh=_sc_mesh(),
        scratch_types=[
            pltpu.VMEM((2, 2, _K), jnp.int32),
            pltpu.VMEM((2, 2, _K), jnp.int32),
            pltpu.VMEM((_K, _D), jnp.float32),
            pltpu.VMEM((_K, _D), jnp.float32),
            pltpu.VMEM_SHARED((_NPAD, _D), jnp.float32),
            pltpu.SemaphoreType.DMA,
            pltpu.SemaphoreType.DMA,
            pltpu.SemaphoreType.DMA,
            pltpu.SemaphoreType.DMA,
        ],
    )


def _mm_body(x_ref, w_ref, o_ref):
    o_ref[...] = jnp.dot(x_ref[...], w_ref[...],
                         precision=lax.Precision.HIGHEST,
                         preferred_element_type=jnp.float32)


def _mm(x, w):
    return pl.pallas_call(
        _mm_body,
        grid=(_GRID,),
        in_specs=[
            pl.BlockSpec((_BLK, _D), lambda i: (i, 0)),
            pl.BlockSpec((_D, _D), lambda i: (0, 0)),
        ],
        out_specs=pl.BlockSpec((_BLK, _D), lambda i: (i, 0)),
        out_shape=jax.ShapeDtypeStruct((_N, _D), jnp.float32),
    )(x, w)


def _disb_of(degp_ref):
    d = degp_ref[0, :, :] + degp_ref[1, :, :] + 1.0
    return jnp.broadcast_to(lax.rsqrt(d), (_BLK, _D))


def _hs_body(degp_ref, h_ref, hs_ref):
    hs_ref[...] = _disb_of(degp_ref) * h_ref[...]


def _hs_mul(degp, h):
    return pl.pallas_call(
        _hs_body,
        grid=(_GRID,),
        in_specs=[
            pl.BlockSpec((_NC, _BLK, 1), lambda i: (0, i, 0)),
            pl.BlockSpec((_BLK, _D), lambda i: (i, 0)),
        ],
        out_specs=pl.BlockSpec((_BLK, _D), lambda i: (i, 0)),
        out_shape=jax.ShapeDtypeStruct((_N, _D), jnp.float32),
    )(degp, h)


def _x1h2_body(agg_ref, degp_ref, b_ref, w_ref, o_ref):
    disb = _disb_of(degp_ref)
    x1 = disb * (agg_ref[0] + agg_ref[1]) + b_ref[...]
    h2 = jnp.dot(x1, w_ref[...], precision=lax.Precision.HIGHEST,
                 preferred_element_type=jnp.float32)
    o_ref[...] = disb * h2


def _x1h2(agg, degp, b, w):
    return pl.pallas_call(
        _x1h2_body,
        grid=(_GRID,),
        in_specs=[
            pl.BlockSpec((_NC, _BLK, _D), lambda i: (0, i, 0)),
            pl.BlockSpec((_NC, _BLK, 1), lambda i: (0, i, 0)),
            pl.BlockSpec((1, _D), lambda i: (0, 0)),
            pl.BlockSpec((_D, _D), lambda i: (0, 0)),
        ],
        out_specs=pl.BlockSpec((_BLK, _D), lambda i: (i, 0)),
        out_shape=jax.ShapeDtypeStruct((_N, _D), jnp.float32),
    )(agg, degp, b, w)


def _final_body(agg_ref, degp_ref, b_ref, o_ref):
    o_ref[...] = (_disb_of(degp_ref) * (agg_ref[0] + agg_ref[1])
                  + b_ref[...])


def _final(agg, degp, b):
    return pl.pallas_call(
        _final_body,
        grid=(_GRID,),
        in_specs=[
            pl.BlockSpec((_NC, _BLK, _D), lambda i: (0, i, 0)),
            pl.BlockSpec((_NC, _BLK, 1), lambda i: (0, i, 0)),
            pl.BlockSpec((1, _D), lambda i: (0, 0)),
        ],
        out_specs=pl.BlockSpec((_BLK, _D), lambda i: (i, 0)),
        out_shape=jax.ShapeDtypeStruct((_N, _D), jnp.float32),
    )(agg, degp, b)


def kernel(adj, in_feat, W1, b1, W2, b2):
    src3 = adj[0].reshape(_NW, _CH, _K)
    dst3 = adj[1].reshape(_NW, _CH, _K)
    idx5 = jnp.stack([src3, dst3], axis=2).reshape(_NW, _CH // 2, 2, 2, _K)
    zeros_1 = jnp.zeros((_NPAD,), jnp.float32)
    ones_k = jnp.ones((_K,), jnp.float32)
    b1r = b1.reshape(1, _D)
    b2r = b2.reshape(1, _D)

    degp = _make_deg()(dst3, ones_k, zeros_1)
    h1 = _mm(in_feat, W1)
    degp = degp.reshape(_NC, _NPAD, 1)
    hs1 = _hs_mul(degp, h1)
    agg1 = _make_agg()(idx5, hs1)
    hs2 = _x1h2(agg1, degp, b1r, W2)
    agg2 = _make_agg()(idx5, hs2)
    return _final(agg2, degp, b2r)


# speedup vs baseline: 1.0214x; 1.0214x over previous
import jax
import jax.numpy as jnp
from jax import lax
from jax.experimental import pallas as pl
from jax.experimental.pallas import tpu as pltpu
from jax.experimental.pallas import tpu_sc as plsc

_N = 10000
_E = 320000
_D = 128
_NPAD = 10240
_NC = 2
_NS = 16
_NW = _NC * _NS
_K = 125
_CH = _E // (_NW * _K)
_RPT = _NPAD // _NS

_BLK = 1000
_GRID = _N // _BLK


def _sc_mesh():
    return plsc.VectorSubcoreMesh(core_axis_name="c", subcore_axis_name="s")


def _deg_body(dst_hbm, ones_hbm, zeros_hbm, out_hbm, dst_v, ones_v, deg_sh):
    cid = lax.axis_index("c")
    sid = lax.axis_index("s")
    wid = sid * _NC + cid
    pltpu.sync_copy(dst_hbm.at[wid], dst_v)
    pltpu.sync_copy(ones_hbm, ones_v)
    rows = pl.ds(sid * _RPT, _RPT)
    pltpu.sync_copy(zeros_hbm.at[rows], deg_sh.at[rows])
    plsc.subcore_barrier()

    @pl.loop(0, _CH)
    def _chunk(j):
        pltpu.sync_copy(ones_v, deg_sh.at[dst_v.at[j]], add=True)

    plsc.subcore_barrier()
    pltpu.sync_copy(deg_sh.at[rows], out_hbm.at[cid, rows])


def _make_deg():
    return pl.kernel(
        _deg_body,
        out_type=jax.ShapeDtypeStruct((_NC, _NPAD), jnp.float32),
        mesh=_sc_mesh(),
        scratch_types=[
            pltpu.VMEM((_CH, _K), jnp.int32),
            pltpu.VMEM((_K,), jnp.float32),
            pltpu.VMEM_SHARED((_NPAD,), jnp.float32),
        ],
    )


def _agg_body(idx_hbm, hs_hbm, out_hbm,
              ibuf_a, ibuf_b, rows_a, rows_b,
              agg_sh, sem_ia, sem_ib, sem_ga, sem_gb):
    cid = lax.axis_index("c")
    sid = lax.axis_index("s")
    wid = sid * _NC + cid
    rows = pl.ds(sid * _RPT, _RPT)

    zv = jnp.zeros((16,), jnp.float32)

    @pl.loop(0, _K)
    def _z(i):
        for c in range(_D // 16):
            rows_a[i, pl.ds(c * 16, 16)] = zv
            rows_b[i, pl.ds(c * 16, 16)] = zv

    base = sid * _RPT

    @pl.when(cid == 0)
    def _():
        @pl.when(sid < _NS - 1)
        def _():
            pltpu.sync_copy(hs_hbm.at[pl.ds(base, _RPT)], agg_sh.at[rows])

        @pl.when(sid == _NS - 1)
        def _():
            pltpu.sync_copy(hs_hbm.at[pl.ds(9600, 400)],
                            agg_sh.at[pl.ds(9600, 400)])
            pltpu.sync_copy(rows_a, agg_sh.at[pl.ds(10000, _K)])
            pltpu.sync_copy(rows_b.at[pl.ds(0, 115)],
                            agg_sh.at[pl.ds(10125, 115)])

    @pl.when(cid != 0)
    def _():
        @pl.loop(0, 5)
        def _f(q):
            pltpu.sync_copy(rows_a, agg_sh.at[pl.ds(base + q * _K, _K)])

        pltpu.sync_copy(rows_b.at[pl.ds(0, 15)],
                        agg_sh.at[pl.ds(base + 5 * _K, 15)])

    plsc.subcore_barrier()

    def start_idx(pair, ibuf, sem):
        pltpu.async_copy(idx_hbm.at[wid, pair], ibuf, sem)

    def wait_idx(pair, ibuf, sem):
        pltpu.make_async_copy(idx_hbm.at[wid, pair], ibuf, sem).wait()

    def start_gather(ibuf, half, buf, sem):
        pltpu.async_copy(hs_hbm.at[ibuf.at[half, 0]], buf, sem)

    def wait_gather(ibuf, half, buf, sem):
        pltpu.make_async_copy(hs_hbm.at[ibuf.at[half, 0]], buf, sem).wait()

    def scatter(buf, ibuf, half):
        pltpu.sync_copy(buf, agg_sh.at[ibuf.at[half, 1]], add=True)

    pltpu.sync_copy(idx_hbm.at[wid, 0], ibuf_a)
    start_gather(ibuf_a, 0, rows_a, sem_ga)
    start_idx(1, ibuf_b, sem_ib)

    @pl.loop(0, _CH, step=4)
    def _quad(j):
        p = j // 4
        wait_gather(ibuf_a, 0, rows_a, sem_ga)
        start_gather(ibuf_a, 1, rows_b, sem_gb)
        scatter(rows_a, ibuf_a, 0)
        wait_idx(2 * p + 1, ibuf_b, sem_ib)
        start_gather(ibuf_b, 0, rows_a, sem_ga)
        wait_gather(ibuf_a, 1, rows_b, sem_gb)
        scatter(rows_b, ibuf_a, 1)

        @pl.when(j + 4 < _CH)
        def _():
            start_idx(2 * p + 2, ibuf_a, sem_ia)

        wait_gather(ibuf_b, 0, rows_a, sem_ga)
        start_gather(ibuf_b, 1, rows_b, sem_gb)
        scatter(rows_a, ibuf_b, 0)

        @pl.when(j + 4 < _CH)
        def _():
            wait_idx(2 * p + 2, ibuf_a, sem_ia)
            start_gather(ibuf_a, 0, rows_a, sem_ga)

        wait_gather(ibuf_b, 1, rows_b, sem_gb)
        scatter(rows_b, ibuf_b, 1)

        @pl.when(j + 6 < _CH)
        def _():
            start_idx(2 * p + 3, ibuf_b, sem_ib)

    plsc.subcore_barrier()
    pltpu.sync_copy(agg_sh.at[rows], out_hbm.at[cid, rows])


def _make_agg():
    return pl.kernel(
        _agg_body,
        out_type=jax.ShapeDtypeStruct((_NC, _NPAD, _D), jnp.float32),
        mesh=_sc_mesh(),
        scratch_types=[
            pltpu.VMEM((2, 2, _K), jnp.int32),
            pltpu.VMEM((2, 2, _K), jnp.int32),
            pltpu.VMEM((_K, _D), jnp.float32),
            pltpu.VMEM((_K, _D), jnp.float32),
            pltpu.VMEM_SHARED((_NPAD, _D), jnp.float32),
            pltpu.SemaphoreType.DMA,
            pltpu.SemaphoreType.DMA,
            pltpu.SemaphoreType.DMA,
            pltpu.SemaphoreType.DMA,
        ],
    )


def _mm_body(x_ref, w_ref, o_ref):
    o_ref[...] = jnp.dot(x_ref[...], w_ref[...],
                         precision=lax.Precision.HIGHEST,
                         preferred_element_type=jnp.float32)


def _mm(x, w):
    return pl.pallas_call(
        _mm_body,
        grid=(_GRID,),
        in_specs=[
            pl.BlockSpec((_BLK, _D), lambda i: (i, 0)),
            pl.BlockSpec((_D, _D), lambda i: (0, 0)),
        ],
        out_specs=pl.BlockSpec((_BLK, _D), lambda i: (i, 0)),
        out_shape=jax.ShapeDtypeStruct((_N, _D), jnp.float32),
    )(x, w)


def _disb_of(degp_ref):
    d = degp_ref[0, :, :] + degp_ref[1, :, :] + 1.0
    return jnp.broadcast_to(lax.rsqrt(d), (_BLK, _D))


def _hs_body(degp_ref, x_ref, w_ref, hs_ref):
    h = jnp.dot(x_ref[...], w_ref[...], precision=lax.Precision.HIGHEST,
                preferred_element_type=jnp.float32)
    hs_ref[...] = _disb_of(degp_ref) * h


def _hs_mul(degp, x, w):
    return pl.pallas_call(
        _hs_body,
        grid=(_GRID,),
        in_specs=[
            pl.BlockSpec((_NC, _BLK, 1), lambda i: (0, i, 0)),
            pl.BlockSpec((_BLK, _D), lambda i: (i, 0)),
            pl.BlockSpec((_D, _D), lambda i: (0, 0)),
        ],
        out_specs=pl.BlockSpec((_BLK, _D), lambda i: (i, 0)),
        out_shape=jax.ShapeDtypeStruct((_N, _D), jnp.float32),
    )(degp, x, w)


def _x1h2_body(agg_ref, degp_ref, b_ref, w_ref, o_ref):
    disb = _disb_of(degp_ref)
    x1 = disb * (agg_ref[0] + agg_ref[1]) + b_ref[...]
    h2 = jnp.dot(x1, w_ref[...], precision=lax.Precision.HIGHEST,
                 preferred_element_type=jnp.float32)
    o_ref[...] = disb * h2


def _x1h2(agg, degp, b, w):
    return pl.pallas_call(
        _x1h2_body,
        grid=(_GRID,),
        in_specs=[
            pl.BlockSpec((_NC, _BLK, _D), lambda i: (0, i, 0)),
            pl.BlockSpec((_NC, _BLK, 1), lambda i: (0, i, 0)),
            pl.BlockSpec((1, _D), lambda i: (0, 0)),
            pl.BlockSpec((_D, _D), lambda i: (0, 0)),
        ],
        out_specs=pl.BlockSpec((_BLK, _D), lambda i: (i, 0)),
        out_shape=jax.ShapeDtypeStruct((_N, _D), jnp.float32),
    )(agg, degp, b, w)


def _final_body(agg_ref, degp_ref, b_ref, o_ref):
    o_ref[...] = (_disb_of(degp_ref) * (agg_ref[0] + agg_ref[1])
                  + b_ref[...])


def _final(agg, degp, b):
    return pl.pallas_call(
        _final_body,
        grid=(_GRID,),
        in_specs=[
            pl.BlockSpec((_NC, _BLK, _D), lambda i: (0, i, 0)),
            pl.BlockSpec((_NC, _BLK, 1), lambda i: (0, i, 0)),
            pl.BlockSpec((1, _D), lambda i: (0, 0)),
        ],
        out_specs=pl.BlockSpec((_BLK, _D), lambda i: (i, 0)),
        out_shape=jax.ShapeDtypeStruct((_N, _D), jnp.float32),
    )(agg, degp, b)


def kernel(adj, in_feat, W1, b1, W2, b2):
    src3 = adj[0].reshape(_NW, _CH, _K)
    dst3 = adj[1].reshape(_NW, _CH, _K)
    idx5 = jnp.stack([src3, dst3], axis=2).reshape(_NW, _CH // 2, 2, 2, _K)
    zeros_1 = jnp.zeros((_NPAD,), jnp.float32)
    ones_k = jnp.ones((_K,), jnp.float32)
    b1r = b1.reshape(1, _D)
    b2r = b2.reshape(1, _D)

    degp = _make_deg()(dst3, ones_k, zeros_1)
    degp = degp.reshape(_NC, _NPAD, 1)
    hs1 = _hs_mul(degp, in_feat, W1)
    agg1 = _make_agg()(idx5, hs1)
    hs2 = _x1h2(agg1, degp, b1r, W2)
    agg2 = _make_agg()(idx5, hs2)
    return _final(agg2, degp, b2r)
